# baseline (speedup 1.0000x reference)
import functools

import jax
import jax.numpy as jnp
from jax import lax
from jax.experimental import pallas as pl
from jax.experimental.pallas import tpu as pltpu
from jax.experimental.pallas import tpu_sc as plsc

D_MODEL = 64
MAXLEN = 200
NUM_CORES = 2
NUM_SUBCORES = 16
NUM_WORKERS = NUM_CORES * NUM_SUBCORES
BPC = 2
HIST = 200
CHUNK = BPC * HIST
NGROUPS = CHUNK // 16


@functools.partial(jax.jit, static_argnames=("bsz",))
def _sc_lookup(positions, table, *, bsz):
    per_w = bsz // NUM_WORKERS
    n_chunks = per_w // BPC
    n_pairs = n_chunks // 2
    mesh = plsc.VectorSubcoreMesh(
        core_axis_name="c", subcore_axis_name="s", num_cores=NUM_CORES
    )

    @functools.partial(
        pl.kernel,
        out_type=jax.ShapeDtypeStruct((bsz, HIST, D_MODEL), jnp.float32),
        mesh=mesh,
        scratch_types=[
            pltpu.VMEM((MAXLEN, D_MODEL), jnp.float32),
            pltpu.VMEM((2, BPC, HIST), jnp.int32),
            pltpu.VMEM((2, BPC, HIST, D_MODEL), jnp.float32),
            pltpu.SemaphoreType.DMA,
            pltpu.SemaphoreType.DMA,
        ],
        compiler_params=pltpu.CompilerParams(
            use_tc_tiling_on_sc=False, needs_layout_passes=False
        ),
    )
    def k(pos_hbm, table_hbm, out_hbm, table_v, idx_v, rows_v, sem_idx, sem_out):
        wid = lax.axis_index("s") * NUM_CORES + lax.axis_index("c")
        b_base = pl.multiple_of(wid * per_w, BPC)

        pltpu.sync_copy(table_hbm, table_v)
        pltpu.sync_copy(pos_hbm.at[pl.ds(b_base, BPC)], idx_v.at[0])
        pltpu.async_copy(
            pos_hbm.at[pl.ds(b_base + BPC, BPC)], idx_v.at[1], sem_idx
        )

        def compute_chunk(h):
            idx_buf = idx_v.at[h]
            rows_buf = rows_v.at[h]

            def group(g, carry):
                r0, r1 = carry
                pos = plsc.load_gather(idx_buf, [r0, r1])
                for d in range(D_MODEL):
                    cd = jnp.full((16,), d, jnp.int32)
                    vals = plsc.load_gather(table_v, [pos, cd])
                    plsc.store_scatter(rows_buf, [r0, r1, cd], vals)
                r1n = r1 + 16
                wrap = r1n >= HIST
                r1n = jnp.where(wrap, r1n - HIST, r1n)
                r0n = jnp.where(wrap, r0 + 1, r0)
                return (r0n, r1n)

            r0_init = jnp.zeros((16,), jnp.int32)
            r1_init = lax.iota(jnp.int32, 16)
            lax.fori_loop(0, NGROUPS, group, (r0_init, r1_init))

        def pair_body(p, _):
            for h in range(2):
                c = 2 * p + h
                b0 = pl.multiple_of(b_base + c * BPC, BPC)
                out_slice = out_hbm.at[pl.ds(b0, BPC)]

                @pl.when(c >= 2)
                def _wait_out():
                    pltpu.make_async_copy(rows_v.at[h], out_slice, sem_out).wait()

                @pl.when(c >= 1)
                def _wait_idx():
                    pltpu.make_async_copy(
                        pos_hbm.at[pl.ds(b0, BPC)], idx_v.at[h], sem_idx
                    ).wait()

                compute_chunk(h)
                pltpu.async_copy(rows_v.at[h], out_slice, sem_out)

                @pl.when(c + 2 < n_chunks)
                def _prefetch_idx():
                    bn = pl.multiple_of(b_base + (c + 2) * BPC, BPC)
                    pltpu.async_copy(
                        pos_hbm.at[pl.ds(bn, BPC)], idx_v.at[h], sem_idx
                    )

            return _

        lax.fori_loop(0, n_pairs, pair_body, 0)

        for h in range(2):
            b0 = pl.multiple_of(b_base + (n_chunks - 2 + h) * BPC, BPC)
            pltpu.make_async_copy(
                rows_v.at[h], out_hbm.at[pl.ds(b0, BPC)], sem_out
            ).wait()

    return k(positions, table)


def kernel(positions, encoding_weight):
    bsz, hist = positions.shape
    maxlen, d = encoding_weight.shape
    assert d == D_MODEL and hist == HIST and maxlen == MAXLEN
    assert bsz % (NUM_WORKERS * BPC * 2) == 0
    return _sc_lookup(positions.astype(jnp.int32), encoding_weight, bsz=bsz)


# speedup vs baseline: 4.0840x; 4.0840x over previous
import functools

import jax
import jax.numpy as jnp
from jax import lax
from jax.experimental import pallas as pl
from jax.experimental.pallas import tpu as pltpu
from jax.experimental.pallas import tpu_sc as plsc

D_MODEL = 64
MAXLEN = 200
NUM_CORES = 2
NUM_SUBCORES = 16
NUM_WORKERS = NUM_CORES * NUM_SUBCORES
BPC = 4
HIST = 200
SPLITS = ((0, 104), (104, 96))


@functools.partial(jax.jit, static_argnames=("bsz",))
def _sc_lookup(positions, table, *, bsz):
    per_w = bsz // NUM_WORKERS
    n_chunks = per_w // BPC
    n_pairs = n_chunks // 2
    rows_bytes = BPC * HIST * D_MODEL * 4
    idx_bytes = BPC * HIST * 4
    mesh = plsc.VectorSubcoreMesh(
        core_axis_name="c", subcore_axis_name="s", num_cores=NUM_CORES
    )

    @functools.partial(
        pl.kernel,
        out_type=jax.ShapeDtypeStruct((bsz, HIST, D_MODEL), jnp.float32),
        mesh=mesh,
        scratch_types=[
            pltpu.VMEM((MAXLEN, D_MODEL), jnp.float32),
            pltpu.VMEM_SHARED((MAXLEN, D_MODEL), jnp.float32),
            pltpu.VMEM((2, BPC, HIST), jnp.int32),
            pltpu.VMEM((2, BPC, HIST, D_MODEL), jnp.float32),
            pltpu.SemaphoreType.DMA,
            pltpu.SemaphoreType.DMA,
            pltpu.SemaphoreType.DMA,
        ],
        compiler_params=pltpu.CompilerParams(use_tc_tiling_on_sc=False),
    )
    def k(pos_hbm, table_hbm, out_hbm, table_v, table_sp, idx_v, rows_v,
          sem_idx, sem_g, sem_out):
        wid = lax.axis_index("s") * NUM_CORES + lax.axis_index("c")
        b_base = pl.multiple_of(wid * per_w, BPC)

        def fire_gathers(h):
            for r in range(BPC):
                for off, ln in SPLITS:
                    pltpu.async_copy(
                        table_sp.at[idx_v.at[h].at[r].at[pl.ds(off, ln)]],
                        rows_v.at[h].at[r].at[pl.ds(off, ln)],
                        sem_g,
                    )

        @pl.when(lax.axis_index("s") == 0)
        def _stage_table():
            pltpu.sync_copy(table_hbm, table_v)
            pltpu.sync_copy(table_v, table_sp)

        plsc.subcore_barrier()
        pltpu.sync_copy(pos_hbm.at[pl.ds(b_base, BPC)], idx_v.at[0])
        pltpu.async_copy(
            pos_hbm.at[pl.ds(b_base + BPC, BPC)], idx_v.at[1], sem_idx
        )
        fire_gathers(0)

        def pair_body(p, _):
            for h in range(2):
                c = 2 * p + h
                b0 = pl.multiple_of(b_base + c * BPC, BPC)
                out_slice = out_hbm.at[pl.ds(b0, BPC)]

                pltpu.make_async_copy(out_slice, rows_v.at[h], sem_g).wait()
                pltpu.async_copy(rows_v.at[h], out_slice, sem_out)

                @pl.when(c >= 1)
                def _wait_out():
                    pltpu.make_async_copy(
                        rows_v.at[1 - h], out_slice, sem_out
                    ).wait()

                @pl.when(c + 1 < n_chunks)
                def _fire_next():
                    pltpu.make_async_copy(
                        pos_hbm.at[pl.ds(b0, BPC)], idx_v.at[1 - h], sem_idx
                    ).wait()
                    fire_gathers(1 - h)

                @pl.when(c + 2 < n_chunks)
                def _prefetch_idx():
                    bn = pl.multiple_of(b_base + (c + 2) * BPC, BPC)
                    pltpu.async_copy(
                        pos_hbm.at[pl.ds(bn, BPC)], idx_v.at[h], sem_idx
                    )

            return _

        lax.fori_loop(0, n_pairs, pair_body, 0)

        last = pl.multiple_of(b_base + (n_chunks - 1) * BPC, BPC)
        pltpu.make_async_copy(
            rows_v.at[(n_chunks - 1) % 2], out_hbm.at[pl.ds(last, BPC)], sem_out
        ).wait()

    return k(positions, table)


def kernel(positions, encoding_weight):
    bsz, hist = positions.shape
    maxlen, d = encoding_weight.shape
    assert d == D_MODEL and hist == HIST and maxlen == MAXLEN
    assert bsz % (NUM_WORKERS * BPC * 2) == 0
    return _sc_lookup(positions.astype(jnp.int32), encoding_weight, bsz=bsz)


# speedup vs baseline: 6.7345x; 1.6490x over previous
import functools

import jax
import jax.numpy as jnp
from jax import lax
from jax.experimental import pallas as pl
from jax.experimental.pallas import tpu as pltpu
from jax.experimental.pallas import tpu_sc as plsc

D_MODEL = 64
MAXLEN = 200
NUM_CORES = 2
NUM_SUBCORES = 16
NUM_WORKERS = NUM_CORES * NUM_SUBCORES
HIST = 200
HIST_PAD = 256
BPB = 64
SPLITS = ((0, 128), (128, 72))


@functools.partial(jax.jit, static_argnames=("bsz",))
def _sc_lookup(pos_flat, table2, *, bsz):
    per_w = bsz // NUM_WORKERS
    n_blocks = per_w // BPB

    mesh = plsc.VectorSubcoreMesh(
        core_axis_name="c", subcore_axis_name="s", num_cores=NUM_CORES
    )

    @functools.partial(
        pl.kernel,
        out_type=jax.ShapeDtypeStruct((bsz, HIST, 128), jnp.float32),
        mesh=mesh,
        scratch_types=[
            pltpu.VMEM((MAXLEN, 128), jnp.float32),
            pltpu.VMEM_SHARED((MAXLEN, 128), jnp.float32),
            pltpu.VMEM((BPB * HIST_PAD,), jnp.int32),
            pltpu.VMEM((2, 1, HIST, 128), jnp.float32),
            pltpu.SemaphoreType.DMA,
            pltpu.SemaphoreType.DMA,
        ],
        compiler_params=pltpu.CompilerParams(use_tc_tiling_on_sc=True),
    )
    def k(pos_hbm, table_hbm, out_hbm, table_v, table_sp, idx_v, rows_v,
          sem_g, sem_out):
        wid = lax.axis_index("s") * NUM_CORES + lax.axis_index("c")
        b_base = wid * per_w
        i_base = pl.multiple_of(b_base * HIST_PAD, 128)

        @pl.when(lax.axis_index("s") == 0)
        def _stage_table():
            pltpu.sync_copy(table_hbm, table_v)
            pltpu.sync_copy(table_v, table_sp)

        plsc.subcore_barrier()

        def fire_gathers(h, rloc):
            for off, ln in SPLITS:
                o = pl.multiple_of(rloc * HIST_PAD + off, 128)
                pltpu.async_copy(
                    table_sp.at[idx_v.at[pl.ds(o, ln)]],
                    rows_v.at[h].at[0].at[pl.ds(off, ln)],
                    sem_g,
                )

        def out_src(h):
            return rows_v.at[h]

        def blk_body(g, _):
            iw = pl.multiple_of(i_base + g * (BPB * HIST_PAD), 128)
            pltpu.sync_copy(pos_hbm.at[pl.ds(iw, BPB * HIST_PAD)], idx_v)
            fire_gathers(0, 0)

            def pair_body(p, _):
                for h in range(2):
                    rloc = 2 * p + h
                    b_abs = b_base + g * BPB + rloc
                    out_slice = out_hbm.at[pl.ds(b_abs, 1)]
                    c_g = g * BPB + rloc

                    pltpu.make_async_copy(
                        table_hbm, rows_v.at[h].at[0], sem_g
                    ).wait()
                    pltpu.async_copy(out_src(h), out_slice, sem_out)

                    @pl.when(c_g >= 1)
                    def _wait_out():
                        pltpu.make_async_copy(
                            out_src(1 - h), out_slice, sem_out
                        ).wait()

                    if h == 1:
                        @pl.when(rloc + 1 < BPB)
                        def _fire_next():
                            fire_gathers(0, rloc + 1)
                    else:
                        fire_gathers(1, rloc + 1)

                return _

            lax.fori_loop(0, BPB // 2, pair_body, 0)
            return _

        lax.fori_loop(0, n_blocks, blk_body, 0)

        last = b_base + per_w - 1
        pltpu.make_async_copy(
            out_src((per_w - 1) % 2), out_hbm.at[pl.ds(last, 1)], sem_out
        ).wait()

    return k(pos_flat, table2)


def kernel(positions, encoding_weight):
    bsz, hist = positions.shape
    maxlen, d = encoding_weight.shape
    assert d == D_MODEL and hist == HIST and maxlen == MAXLEN
    assert bsz % (NUM_WORKERS * BPB) == 0
    pos_pad = jnp.pad(positions.astype(jnp.int32), ((0, 0), (0, HIST_PAD - hist)))
    pos_flat = pos_pad.reshape(-1)
    table2 = jnp.concatenate([encoding_weight, encoding_weight], axis=1)
    return _sc_lookup(pos_flat, table2, bsz=bsz)[:, :, :D_MODEL]
